# baseline (speedup 1.0000x reference)
import functools

import jax
import jax.numpy as jnp
from jax import lax
from jax.experimental import pallas as pl
from jax.experimental.pallas import tpu as pltpu
from jax.experimental.pallas import tpu_sc as plsc

B = 16384
F = 32
NC = 2
NS = 16
NW = NC * NS
BPW = B // NW
CHUNK = 128
NCH = BPW // CHUNK

_mesh = plsc.VectorSubcoreMesh(core_axis_name="c", subcore_axis_name="s")


@functools.partial(
    pl.kernel,
    mesh=_mesh,
    out_type=jax.ShapeDtypeStruct((B,), jnp.float32),
    compiler_params=pltpu.CompilerParams(
        needs_layout_passes=False, use_tc_tiling_on_sc=False),
    scratch_types=[
        pltpu.VMEM((NCH, CHUNK), jnp.int32),
        pltpu.VMEM((NCH, CHUNK), jnp.int32),
        pltpu.VMEM((F, BPW), jnp.float32),
        pltpu.VMEM((F, BPW), jnp.float32),
        pltpu.VMEM((BPW,), jnp.float32),
        pltpu.SemaphoreType.DMA,
        pltpu.SemaphoreType.DMA,
    ],
)
def _mf_sc(user_hbm, item_hbm, uft_hbm, ift_hbm, out_hbm,
           uidx, iidx, ubuf, ibuf, outv, sem_u, sem_i):
    wid = lax.axis_index("s") * NC + lax.axis_index("c")
    base = wid * BPW

    for j in range(NCH):
        pltpu.sync_copy(user_hbm.at[pl.ds(base + j * CHUNK, CHUNK)], uidx.at[j])
        pltpu.sync_copy(item_hbm.at[pl.ds(base + j * CHUNK, CHUNK)], iidx.at[j])

    copies = []
    for f in range(F):
        for j in range(NCH):
            copies.append(pltpu.async_copy(
                uft_hbm.at[f].at[uidx.at[j]],
                ubuf.at[f, pl.ds(j * CHUNK, CHUNK)], sem_u))
            copies.append(pltpu.async_copy(
                ift_hbm.at[f].at[iidx.at[j]],
                ibuf.at[f, pl.ds(j * CHUNK, CHUNK)], sem_i))
    for c in copies:
        c.wait()

    def body(g, carry):
        s = pl.ds(g * 16, 16)
        acc = jnp.zeros((16,), jnp.float32)
        for f in range(F):
            acc = acc + ubuf[f, s] * ibuf[f, s]
        outv[s] = acc
        return carry

    lax.fori_loop(0, BPW // 16, body, 0)

    pltpu.sync_copy(outv, out_hbm.at[pl.ds(base, BPW)])


def kernel(user, item, user_factors, item_factors):
    return _mf_sc(user, item, user_factors.T, item_factors.T)


# speedup vs baseline: 8.0717x; 8.0717x over previous
import functools

import jax
import jax.numpy as jnp
from jax import lax
from jax.experimental import pallas as pl
from jax.experimental.pallas import tpu as pltpu
from jax.experimental.pallas import tpu_sc as plsc

B = 16384
F = 32
NC = 2
NS = 16
NW = NC * NS
BPW = B // NW
CHUNK = 128
NCH = BPW // CHUNK
GRP = CHUNK // 16

BLK = 8192


QTR = BLK // 4


def _repack_body(src_ref, dst_ref):
    x = src_ref[...]
    y = jnp.transpose(x)
    dst_ref[...] = jnp.concatenate(
        [y[q * QTR:(q + 1) * QTR] for q in range(4)], axis=1)


def _repack(table_t):
    f, n = table_t.shape
    grid = (n + BLK - 1) // BLK
    return pl.pallas_call(
        _repack_body,
        grid=(grid,),
        in_specs=[pl.BlockSpec((F, BLK), lambda i: (0, i))],
        out_specs=pl.BlockSpec((QTR, 128), lambda i: (i, 0)),
        out_shape=jax.ShapeDtypeStruct((grid * QTR, 128), jnp.float32),
    )(table_t)


_mesh = plsc.VectorSubcoreMesh(core_axis_name="c", subcore_axis_name="s")


@functools.partial(
    pl.kernel,
    mesh=_mesh,
    out_type=jax.ShapeDtypeStruct((B,), jnp.float32),
    compiler_params=pltpu.CompilerParams(needs_layout_passes=False),
    scratch_types=[
        pltpu.VMEM((NCH, CHUNK), jnp.int32),
        pltpu.VMEM((NCH, CHUNK), jnp.int32),
        pltpu.VMEM((NCH, CHUNK), jnp.int32),
        pltpu.VMEM((NCH, CHUNK), jnp.int32),
        pltpu.VMEM((CHUNK, 128), jnp.float32),
        pltpu.VMEM((CHUNK, 128), jnp.float32),
        pltpu.VMEM((BPW,), jnp.float32),
        pltpu.SemaphoreType.DMA,
        pltpu.SemaphoreType.DMA,
    ],
)
def _mf_sc(user_hbm, item_hbm, uf_hbm, if_hbm, out_hbm,
           uidx, iidx, urow, irow, ubuf, ibuf, outv, sem_u, sem_i):
    wid = lax.axis_index("s") * NC + lax.axis_index("c")
    base = wid * BPW

    for j in range(NCH):
        pltpu.sync_copy(user_hbm.at[pl.ds(base + j * CHUNK, CHUNK)], uidx.at[j])
        pltpu.sync_copy(item_hbm.at[pl.ds(base + j * CHUNK, CHUNK)], iidx.at[j])
        for g in range(GRP):
            s = pl.ds(g * 16, 16)
            u = uidx[j, s]
            i = iidx[j, s]
            urow[j, s] = lax.shift_left(
                lax.shift_right_logical(u, 13), 11) + jnp.bitwise_and(u, 2047)
            irow[j, s] = lax.shift_left(
                lax.shift_right_logical(i, 13), 11) + jnp.bitwise_and(i, 2047)

    for j in range(NCH):
        cu = pltpu.async_copy(uf_hbm.at[urow.at[j]], ubuf, sem_u)
        ci = pltpu.async_copy(if_hbm.at[irow.at[j]], ibuf, sem_i)
        cu.wait()
        ci.wait()

        def body(g, carry):
            rows = g * 16 + lax.iota(jnp.int32, 16)
            s = pl.ds(g * 16, 16)
            ucol = lax.shift_left(
                jnp.bitwise_and(lax.shift_right_logical(uidx[j, s], 11), 3), 5)
            icol = lax.shift_left(
                jnp.bitwise_and(lax.shift_right_logical(iidx[j, s], 11), 3), 5)
            acc = jnp.zeros((16,), jnp.float32)
            for f in range(F):
                gu = plsc.load_gather(ubuf, [rows, ucol + f])
                gi = plsc.load_gather(ibuf, [rows, icol + f])
                acc = acc + gu * gi
            outv[pl.ds(j * CHUNK + g * 16, 16)] = acc
            return carry

        lax.fori_loop(0, GRP, body, 0)

    pltpu.sync_copy(outv, out_hbm.at[pl.ds(base, BPW)])


def kernel(user, item, user_factors, item_factors):
    uf128 = _repack(user_factors.T)
    if128 = _repack(item_factors.T)
    return _mf_sc(user, item, uf128, if128)


# speedup vs baseline: 13.2171x; 1.6375x over previous
import functools

import jax
import jax.numpy as jnp
from jax import lax
from jax.experimental import pallas as pl
from jax.experimental.pallas import tpu as pltpu
from jax.experimental.pallas import tpu_sc as plsc

B = 16384
F = 32
NC = 2
NS = 16
NW = NC * NS
BPW = B // NW
CHUNK = 128
NCH = BPW // CHUNK
GRP = CHUNK // 16

BLK = 8192


QTR = BLK // 4


def _repack_body(src_ref, dst_ref):
    for s in range(BLK // 512):
        z = jnp.concatenate(
            [src_ref[:, pl.ds(512 * s + 128 * g, 128)] for g in range(4)],
            axis=0)
        dst_ref[pl.ds(s * 128, 128), :] = jnp.transpose(z)


def _repack(table_t):
    f, n = table_t.shape
    grid = (n + BLK - 1) // BLK
    return pl.pallas_call(
        _repack_body,
        grid=(grid,),
        in_specs=[pl.BlockSpec((F, BLK), lambda i: (0, i))],
        out_specs=pl.BlockSpec((QTR, 128), lambda i: (i, 0)),
        out_shape=jax.ShapeDtypeStruct((grid * QTR, 128), jnp.float32),
    )(table_t)


_mesh = plsc.VectorSubcoreMesh(core_axis_name="c", subcore_axis_name="s")


@functools.partial(
    pl.kernel,
    mesh=_mesh,
    out_type=jax.ShapeDtypeStruct((B,), jnp.float32),
    compiler_params=pltpu.CompilerParams(needs_layout_passes=False),
    scratch_types=[
        pltpu.VMEM((NCH, CHUNK), jnp.int32),
        pltpu.VMEM((NCH, CHUNK), jnp.int32),
        pltpu.VMEM((NCH, CHUNK), jnp.int32),
        pltpu.VMEM((NCH, CHUNK), jnp.int32),
        pltpu.VMEM((CHUNK, 128), jnp.float32),
        pltpu.VMEM((CHUNK, 128), jnp.float32),
        pltpu.VMEM((BPW,), jnp.float32),
        pltpu.SemaphoreType.DMA,
        pltpu.SemaphoreType.DMA,
    ],
)
def _mf_sc(user_hbm, item_hbm, uf_hbm, if_hbm, out_hbm,
           uidx, iidx, urow, irow, ubuf, ibuf, outv, sem_u, sem_i):
    wid = lax.axis_index("s") * NC + lax.axis_index("c")
    base = wid * BPW

    for j in range(NCH):
        pltpu.sync_copy(user_hbm.at[pl.ds(base + j * CHUNK, CHUNK)], uidx.at[j])
        pltpu.sync_copy(item_hbm.at[pl.ds(base + j * CHUNK, CHUNK)], iidx.at[j])
        for g in range(GRP):
            s = pl.ds(g * 16, 16)
            u = uidx[j, s]
            i = iidx[j, s]
            urow[j, s] = lax.shift_left(
                lax.shift_right_logical(u, 9), 7) + jnp.bitwise_and(u, 127)
            irow[j, s] = lax.shift_left(
                lax.shift_right_logical(i, 9), 7) + jnp.bitwise_and(i, 127)

    for j in range(NCH):
        cu = pltpu.async_copy(uf_hbm.at[urow.at[j]], ubuf, sem_u)
        ci = pltpu.async_copy(if_hbm.at[irow.at[j]], ibuf, sem_i)
        cu.wait()
        ci.wait()

        def body(g, carry):
            rows = g * 16 + lax.iota(jnp.int32, 16)
            s = pl.ds(g * 16, 16)
            ucol = lax.shift_left(
                jnp.bitwise_and(lax.shift_right_logical(uidx[j, s], 7), 3), 5)
            icol = lax.shift_left(
                jnp.bitwise_and(lax.shift_right_logical(iidx[j, s], 7), 3), 5)
            acc = jnp.zeros((16,), jnp.float32)
            for f in range(F):
                gu = plsc.load_gather(ubuf, [rows, ucol + f])
                gi = plsc.load_gather(ibuf, [rows, icol + f])
                acc = acc + gu * gi
            outv[pl.ds(j * CHUNK + g * 16, 16)] = acc
            return carry

        lax.fori_loop(0, GRP, body, 0)

    pltpu.sync_copy(outv, out_hbm.at[pl.ds(base, BPW)])


def kernel(user, item, user_factors, item_factors):
    uf128 = _repack(user_factors.T)
    if128 = _repack(item_factors.T)
    return _mf_sc(user, item, uf128, if128)


# speedup vs baseline: 18.2859x; 1.3835x over previous
import functools

import jax
import jax.numpy as jnp
from jax import lax
from jax.experimental import pallas as pl
from jax.experimental.pallas import tpu as pltpu
from jax.experimental.pallas import tpu_sc as plsc

B = 16384
F = 32
NC = 2
NS = 16
NW = NC * NS
BPW = B // NW
CHUNK = 128
NCH = BPW // CHUNK
GRP = CHUNK // 16

BLK = 32768


QTR = BLK // 4


def _repack_body(src_ref, dst_ref):
    for s in range(BLK // 512):
        z = jnp.concatenate(
            [src_ref[:, pl.ds(512 * s + 128 * g, 128)] for g in range(4)],
            axis=0)
        dst_ref[pl.ds(s * 128, 128), :] = jnp.transpose(z)


def _repack(table_t):
    f, n = table_t.shape
    grid = (n + BLK - 1) // BLK
    return pl.pallas_call(
        _repack_body,
        grid=(grid,),
        in_specs=[pl.BlockSpec((F, BLK), lambda i: (0, i))],
        out_specs=pl.BlockSpec((QTR, 128), lambda i: (i, 0)),
        out_shape=jax.ShapeDtypeStruct((grid * QTR, 128), jnp.float32),
    )(table_t)


_mesh = plsc.VectorSubcoreMesh(core_axis_name="c", subcore_axis_name="s")


@functools.partial(
    pl.kernel,
    mesh=_mesh,
    out_type=jax.ShapeDtypeStruct((B,), jnp.float32),
    compiler_params=pltpu.CompilerParams(needs_layout_passes=False),
    scratch_types=[
        pltpu.VMEM((NCH, CHUNK), jnp.int32),
        pltpu.VMEM((NCH, CHUNK), jnp.int32),
        pltpu.VMEM((NCH, CHUNK), jnp.int32),
        pltpu.VMEM((NCH, CHUNK), jnp.int32),
        pltpu.VMEM((CHUNK, 128), jnp.float32),
        pltpu.VMEM((CHUNK, 128), jnp.float32),
        pltpu.VMEM((BPW,), jnp.float32),
        pltpu.SemaphoreType.DMA,
        pltpu.SemaphoreType.DMA,
    ],
)
def _mf_sc(user_hbm, item_hbm, uf_hbm, if_hbm, out_hbm,
           uidx, iidx, urow, irow, ubuf, ibuf, outv, sem_u, sem_i):
    wid = lax.axis_index("s") * NC + lax.axis_index("c")
    base = wid * BPW

    for j in range(NCH):
        pltpu.sync_copy(user_hbm.at[pl.ds(base + j * CHUNK, CHUNK)], uidx.at[j])
        pltpu.sync_copy(item_hbm.at[pl.ds(base + j * CHUNK, CHUNK)], iidx.at[j])
        for g in range(GRP):
            s = pl.ds(g * 16, 16)
            u = uidx[j, s]
            i = iidx[j, s]
            urow[j, s] = lax.shift_left(
                lax.shift_right_logical(u, 9), 7) + jnp.bitwise_and(u, 127)
            irow[j, s] = lax.shift_left(
                lax.shift_right_logical(i, 9), 7) + jnp.bitwise_and(i, 127)

    for j in range(NCH):
        cu = pltpu.async_copy(uf_hbm.at[urow.at[j]], ubuf, sem_u)
        ci = pltpu.async_copy(if_hbm.at[irow.at[j]], ibuf, sem_i)
        cu.wait()
        ci.wait()

        def body(g, carry):
            rows = g * 16 + lax.iota(jnp.int32, 16)
            s = pl.ds(g * 16, 16)
            ucol = lax.shift_left(
                jnp.bitwise_and(lax.shift_right_logical(uidx[j, s], 7), 3), 5)
            icol = lax.shift_left(
                jnp.bitwise_and(lax.shift_right_logical(iidx[j, s], 7), 3), 5)
            acc = jnp.zeros((16,), jnp.float32)
            for f in range(F):
                gu = plsc.load_gather(ubuf, [rows, ucol + f])
                gi = plsc.load_gather(ibuf, [rows, icol + f])
                acc = acc + gu * gi
            outv[pl.ds(j * CHUNK + g * 16, 16)] = acc
            return carry

        lax.fori_loop(0, GRP, body, 0)

    pltpu.sync_copy(outv, out_hbm.at[pl.ds(base, BPW)])


def kernel(user, item, user_factors, item_factors):
    uf128 = _repack(user_factors.T)
    if128 = _repack(item_factors.T)
    return _mf_sc(user, item, uf128, if128)


# speedup vs baseline: 18.4825x; 1.0108x over previous
import functools

import jax
import jax.numpy as jnp
from jax import lax
from jax.experimental import pallas as pl
from jax.experimental.pallas import tpu as pltpu
from jax.experimental.pallas import tpu_sc as plsc

B = 16384
F = 32
NC = 2
NS = 16
NW = NC * NS
BPW = B // NW
CHUNK = 128
NCH = BPW // CHUNK
GRP = CHUNK // 16

BLK = 65536


QTR = BLK // 4


def _repack_body(src_ref, dst_ref):
    for s in range(BLK // 512):
        z = jnp.concatenate(
            [src_ref[:, pl.ds(512 * s + 128 * g, 128)] for g in range(4)],
            axis=0)
        dst_ref[pl.ds(s * 128, 128), :] = jnp.transpose(z)


def _repack(table_t):
    f, n = table_t.shape
    grid = (n + BLK - 1) // BLK
    return pl.pallas_call(
        _repack_body,
        grid=(grid,),
        in_specs=[pl.BlockSpec((F, BLK), lambda i: (0, i))],
        out_specs=pl.BlockSpec((QTR, 128), lambda i: (i, 0)),
        out_shape=jax.ShapeDtypeStruct((grid * QTR, 128), jnp.float32),
    )(table_t)


_mesh = plsc.VectorSubcoreMesh(core_axis_name="c", subcore_axis_name="s")


@functools.partial(
    pl.kernel,
    mesh=_mesh,
    out_type=jax.ShapeDtypeStruct((B,), jnp.float32),
    compiler_params=pltpu.CompilerParams(needs_layout_passes=False),
    scratch_types=[
        pltpu.VMEM((NCH, CHUNK), jnp.int32),
        pltpu.VMEM((NCH, CHUNK), jnp.int32),
        pltpu.VMEM((NCH, CHUNK), jnp.int32),
        pltpu.VMEM((NCH, CHUNK), jnp.int32),
        pltpu.VMEM((CHUNK, 128), jnp.float32),
        pltpu.VMEM((CHUNK, 128), jnp.float32),
        pltpu.VMEM((BPW,), jnp.float32),
        pltpu.SemaphoreType.DMA,
        pltpu.SemaphoreType.DMA,
    ],
)
def _mf_sc(user_hbm, item_hbm, uf_hbm, if_hbm, out_hbm,
           uidx, iidx, urow, irow, ubuf, ibuf, outv, sem_u, sem_i):
    wid = lax.axis_index("s") * NC + lax.axis_index("c")
    base = wid * BPW

    for j in range(NCH):
        pltpu.sync_copy(user_hbm.at[pl.ds(base + j * CHUNK, CHUNK)], uidx.at[j])
        pltpu.sync_copy(item_hbm.at[pl.ds(base + j * CHUNK, CHUNK)], iidx.at[j])
        for g in range(GRP):
            s = pl.ds(g * 16, 16)
            u = uidx[j, s]
            i = iidx[j, s]
            urow[j, s] = lax.shift_left(
                lax.shift_right_logical(u, 9), 7) + jnp.bitwise_and(u, 127)
            irow[j, s] = lax.shift_left(
                lax.shift_right_logical(i, 9), 7) + jnp.bitwise_and(i, 127)

    for j in range(NCH):
        cu = pltpu.async_copy(uf_hbm.at[urow.at[j]], ubuf, sem_u)
        ci = pltpu.async_copy(if_hbm.at[irow.at[j]], ibuf, sem_i)
        cu.wait()
        ci.wait()

        def body(g, carry):
            rows = g * 16 + lax.iota(jnp.int32, 16)
            s = pl.ds(g * 16, 16)
            ucol = lax.shift_left(
                jnp.bitwise_and(lax.shift_right_logical(uidx[j, s], 7), 3), 5)
            icol = lax.shift_left(
                jnp.bitwise_and(lax.shift_right_logical(iidx[j, s], 7), 3), 5)
            acc = jnp.zeros((16,), jnp.float32)
            for f in range(F):
                gu = plsc.load_gather(ubuf, [rows, ucol + f])
                gi = plsc.load_gather(ibuf, [rows, icol + f])
                acc = acc + gu * gi
            outv[pl.ds(j * CHUNK + g * 16, 16)] = acc
            return carry

        lax.fori_loop(0, GRP, body, 0)

    pltpu.sync_copy(outv, out_hbm.at[pl.ds(base, BPW)])


def kernel(user, item, user_factors, item_factors):
    uf128 = _repack(user_factors.T)
    if128 = _repack(item_factors.T)
    return _mf_sc(user, item, uf128, if128)


# speedup vs baseline: 19.1488x; 1.0360x over previous
import functools

import jax
import jax.numpy as jnp
from jax import lax
from jax.experimental import pallas as pl
from jax.experimental.pallas import tpu as pltpu
from jax.experimental.pallas import tpu_sc as plsc

B = 16384
F = 32
NC = 2
NS = 16
NW = NC * NS
BPW = B // NW
CHUNK = 128
NCH = BPW // CHUNK
GRP = CHUNK // 16

BLK = 65536


QTR = BLK // 4


def _repack_body(src_ref, dst_ref):
    for s in range(BLK // 512):
        z = jnp.concatenate(
            [src_ref[:, pl.ds(512 * s + 128 * g, 128)] for g in range(4)],
            axis=0)
        dst_ref[pl.ds(s * 128, 128), :] = jnp.transpose(z)


def _repack(table_t):
    f, n = table_t.shape
    grid = (n + BLK - 1) // BLK
    return pl.pallas_call(
        _repack_body,
        grid=(grid,),
        in_specs=[pl.BlockSpec((F, BLK), lambda i: (0, i))],
        out_specs=pl.BlockSpec((QTR, 128), lambda i: (i, 0)),
        out_shape=jax.ShapeDtypeStruct((grid * QTR, 128), jnp.float32),
    )(table_t)


_mesh = plsc.VectorSubcoreMesh(core_axis_name="c", subcore_axis_name="s")


@functools.partial(
    pl.kernel,
    mesh=_mesh,
    out_type=jax.ShapeDtypeStruct((B,), jnp.float32),
    compiler_params=pltpu.CompilerParams(needs_layout_passes=False),
    scratch_types=[
        pltpu.VMEM((NCH, CHUNK), jnp.int32),
        pltpu.VMEM((NCH, CHUNK), jnp.int32),
        pltpu.VMEM((NCH, CHUNK), jnp.int32),
        pltpu.VMEM((NCH, CHUNK), jnp.int32),
        pltpu.VMEM((2, CHUNK, 128), jnp.float32),
        pltpu.VMEM((2, CHUNK, 128), jnp.float32),
        pltpu.VMEM((BPW,), jnp.float32),
        pltpu.SemaphoreType.DMA,
        pltpu.SemaphoreType.DMA,
        pltpu.SemaphoreType.DMA,
        pltpu.SemaphoreType.DMA,
    ],
)
def _mf_sc(user_hbm, item_hbm, uf_hbm, if_hbm, out_hbm,
           uidx, iidx, urow, irow, ubuf, ibuf, outv,
           sem_u0, sem_u1, sem_i0, sem_i1):
    wid = lax.axis_index("s") * NC + lax.axis_index("c")
    base = wid * BPW

    for j in range(NCH):
        pltpu.sync_copy(user_hbm.at[pl.ds(base + j * CHUNK, CHUNK)], uidx.at[j])
        pltpu.sync_copy(item_hbm.at[pl.ds(base + j * CHUNK, CHUNK)], iidx.at[j])
        for g in range(GRP):
            s = pl.ds(g * 16, 16)
            u = uidx[j, s]
            i = iidx[j, s]
            urow[j, s] = lax.shift_left(
                lax.shift_right_logical(u, 9), 7) + jnp.bitwise_and(u, 127)
            irow[j, s] = lax.shift_left(
                lax.shift_right_logical(i, 9), 7) + jnp.bitwise_and(i, 127)

    sems_u = (sem_u0, sem_u1)
    sems_i = (sem_i0, sem_i1)

    def fire(j):
        p = j % 2
        cu = pltpu.async_copy(uf_hbm.at[urow.at[j]], ubuf.at[p], sems_u[p])
        ci = pltpu.async_copy(if_hbm.at[irow.at[j]], ibuf.at[p], sems_i[p])
        return cu, ci

    pending = fire(0)
    for j in range(NCH):
        nxt = fire(j + 1) if j + 1 < NCH else None
        cu, ci = pending
        cu.wait()
        ci.wait()
        pending = nxt
        p = j % 2

        def body(g, carry):
            rows = g * 16 + lax.iota(jnp.int32, 16)
            s = pl.ds(g * 16, 16)
            ucol = lax.shift_left(
                jnp.bitwise_and(lax.shift_right_logical(uidx[j, s], 7), 3), 5)
            icol = lax.shift_left(
                jnp.bitwise_and(lax.shift_right_logical(iidx[j, s], 7), 3), 5)
            acc = jnp.zeros((16,), jnp.float32)
            for f in range(F):
                gu = plsc.load_gather(ubuf.at[p], [rows, ucol + f])
                gi = plsc.load_gather(ibuf.at[p], [rows, icol + f])
                acc = acc + gu * gi
            outv[pl.ds(j * CHUNK + g * 16, 16)] = acc
            return carry

        lax.fori_loop(0, GRP, body, 0)

    pltpu.sync_copy(outv, out_hbm.at[pl.ds(base, BPW)])


def kernel(user, item, user_factors, item_factors):
    uf128 = _repack(user_factors.T)
    if128 = _repack(item_factors.T)
    return _mf_sc(user, item, uf128, if128)


# speedup vs baseline: 19.6787x; 1.0277x over previous
import functools

import jax
import jax.numpy as jnp
from jax import lax
from jax.experimental import pallas as pl
from jax.experimental.pallas import tpu as pltpu
from jax.experimental.pallas import tpu_sc as plsc

B = 16384
F = 32
NC = 2
NS = 16
NW = NC * NS
BPW = B // NW
CHUNK = 128
NCH = BPW // CHUNK
GRP = CHUNK // 16

BLK = 65536


QTR = BLK // 4


def _repack_body(src_ref, dst_ref):
    for s in range(BLK // 512):
        z = jnp.concatenate(
            [src_ref[:, pl.ds(512 * s + 128 * g, 128)] for g in range(4)],
            axis=0)
        dst_ref[pl.ds(s * 128, 128), :] = jnp.transpose(z)


def _repack(table_t):
    f, n = table_t.shape
    grid = (n + BLK - 1) // BLK
    return pl.pallas_call(
        _repack_body,
        grid=(grid,),
        in_specs=[pl.BlockSpec((F, BLK), lambda i: (0, i))],
        out_specs=pl.BlockSpec((QTR, 128), lambda i: (i, 0)),
        out_shape=jax.ShapeDtypeStruct((grid * QTR, 128), jnp.float32),
    )(table_t)


_mesh = plsc.VectorSubcoreMesh(core_axis_name="c", subcore_axis_name="s")


@functools.partial(
    pl.kernel,
    mesh=_mesh,
    out_type=jax.ShapeDtypeStruct((B,), jnp.float32),
    compiler_params=pltpu.CompilerParams(needs_layout_passes=False),
    scratch_types=[
        pltpu.VMEM((NCH, CHUNK), jnp.int32),
        pltpu.VMEM((NCH, CHUNK), jnp.int32),
        pltpu.VMEM((NCH, CHUNK), jnp.int32),
        pltpu.VMEM((NCH, CHUNK), jnp.int32),
        pltpu.VMEM((2, CHUNK, 128), jnp.float32),
        pltpu.VMEM((2, CHUNK, 128), jnp.float32),
        pltpu.VMEM((BPW,), jnp.float32),
        pltpu.SemaphoreType.DMA,
        pltpu.SemaphoreType.DMA,
        pltpu.SemaphoreType.DMA,
        pltpu.SemaphoreType.DMA,
    ],
)
def _mf_sc(user_hbm, item_hbm, uf_hbm, if_hbm, out_hbm,
           uidx, iidx, urow, irow, ubuf, ibuf, outv,
           sem_u0, sem_u1, sem_i0, sem_i1):
    wid = lax.axis_index("s") * NC + lax.axis_index("c")
    base = wid * BPW

    idx_copies = []
    for j in range(NCH):
        idx_copies.append(pltpu.async_copy(
            user_hbm.at[pl.ds(base + j * CHUNK, CHUNK)], uidx.at[j], sem_u0))
        idx_copies.append(pltpu.async_copy(
            item_hbm.at[pl.ds(base + j * CHUNK, CHUNK)], iidx.at[j], sem_i0))
    for c in idx_copies:
        c.wait()
    for j in range(NCH):
        for g in range(GRP):
            s = pl.ds(g * 16, 16)
            u = uidx[j, s]
            i = iidx[j, s]
            urow[j, s] = lax.shift_left(
                lax.shift_right_logical(u, 9), 7) + jnp.bitwise_and(u, 127)
            irow[j, s] = lax.shift_left(
                lax.shift_right_logical(i, 9), 7) + jnp.bitwise_and(i, 127)

    sems_u = (sem_u0, sem_u1)
    sems_i = (sem_i0, sem_i1)

    def fire(j):
        p = j % 2
        cu = pltpu.async_copy(uf_hbm.at[urow.at[j]], ubuf.at[p], sems_u[p])
        ci = pltpu.async_copy(if_hbm.at[irow.at[j]], ibuf.at[p], sems_i[p])
        return cu, ci

    pending = fire(0)
    for j in range(NCH):
        nxt = fire(j + 1) if j + 1 < NCH else None
        cu, ci = pending
        cu.wait()
        ci.wait()
        pending = nxt
        p = j % 2

        def body(g, carry):
            rows = g * 16 + lax.iota(jnp.int32, 16)
            s = pl.ds(g * 16, 16)
            ucol = lax.shift_left(
                jnp.bitwise_and(lax.shift_right_logical(uidx[j, s], 7), 3), 5)
            icol = lax.shift_left(
                jnp.bitwise_and(lax.shift_right_logical(iidx[j, s], 7), 3), 5)
            acc = jnp.zeros((16,), jnp.float32)
            for f in range(F):
                gu = plsc.load_gather(ubuf.at[p], [rows, ucol + f])
                gi = plsc.load_gather(ibuf.at[p], [rows, icol + f])
                acc = acc + gu * gi
            outv[pl.ds(j * CHUNK + g * 16, 16)] = acc
            return carry

        lax.fori_loop(0, GRP, body, 0)

    pltpu.sync_copy(outv, out_hbm.at[pl.ds(base, BPW)])


def kernel(user, item, user_factors, item_factors):
    uf128 = _repack(user_factors.T)
    if128 = _repack(item_factors.T)
    return _mf_sc(user, item, uf128, if128)
